# baseline (speedup 1.0000x reference)
import jax
import jax.numpy as jnp
from jax import lax
from jax.experimental import pallas as pl
from jax.experimental.pallas import tpu as pltpu
from jax.experimental.pallas import tpu_sc as plsc

B = 16
N = 16384
K = 2048
L = 16
HALF = N // 2
HCHUNKS = HALF // L


def _fps_body(points_hbm, out_hbm,
              x_v, y_v, z_v, d_v, ox_v, oy_v, oz_v, comm_v, pr_v, sh_s):
    c = lax.axis_index("c")
    s = lax.axis_index("s")
    b = c * 8 + s // 2
    half = s % 2
    partner = s ^ 1
    hbase = half * HALF

    pltpu.sync_copy(points_hbm.at[pl.ds((b * 3 + 0) * N, N)], x_v)
    pltpu.sync_copy(points_hbm.at[pl.ds((b * 3 + 1) * N, N)], y_v)
    pltpu.sync_copy(points_hbm.at[pl.ds((b * 3 + 2) * N, N)], z_v)

    lanes = lax.iota(jnp.int32, L)
    lane0 = lanes == 0
    big = jnp.full((L,), 1e10, jnp.float32)

    @plsc.parallel_loop(0, HCHUNKS, unroll=8)
    def _init(i):
        d_v[pl.ds(i * L, L)] = big

    zero_idx = jnp.full((L,), jnp.min(lanes), jnp.int32)
    lx = plsc.load_gather(x_v, [zero_idx])
    ly = plsc.load_gather(y_v, [zero_idx])
    lz = plsc.load_gather(z_v, [zero_idx])
    plsc.store_scatter(ox_v, [zero_idx], lx, mask=lane0)
    plsc.store_scatter(oy_v, [zero_idx], ly, mask=lane0)
    plsc.store_scatter(oz_v, [zero_idx], lz, mask=lane0)

    eq0 = lanes == 0
    eq1 = lanes == 1
    eq2 = lanes == 2
    eq3 = lanes == 3
    one_idx = jnp.full((L,), 1, jnp.int32)
    two_idx = jnp.full((L,), 2, jnp.int32)
    three_idx = jnp.full((L,), 3, jnp.int32)
    four_idx = jnp.full((L,), 4, jnp.int32)

    def outer(k, carry):
        lx, ly, lz = carry

        def inner(i, mcarry):
            run_max, run_idx = mcarry
            base = i * L
            xv = x_v[pl.ds(hbase + base, L)]
            yv = y_v[pl.ds(hbase + base, L)]
            zv = z_v[pl.ds(hbase + base, L)]
            dv = d_v[pl.ds(base, L)]
            dx = xv - lx
            dy = yv - ly
            dz = zv - lz
            dist = (dx * dx + dz * dz) + dy * dy
            nd = jnp.minimum(dv, dist)
            d_v[pl.ds(base, L)] = nd
            upd = nd > run_max
            run_max = jnp.where(upd, nd, run_max)
            run_idx = jnp.where(upd, hbase + base + lanes, run_idx)
            return run_max, run_idx

        run_max0 = jnp.full((L,), -jnp.inf, jnp.float32)
        run_idx0 = jnp.zeros((L,), jnp.int32)
        run_max, run_idx = plsc.parallel_loop(
            0, HCHUNKS, carry=(run_max0, run_idx0), unroll=8
        )(inner)
        m = jnp.max(run_max)
        sel = jnp.min(jnp.where(run_max == m, run_idx, N))
        mv = jnp.full((L,), m, jnp.float32)
        selv = jnp.full((L,), sel, jnp.int32)
        gx = plsc.load_gather(x_v, [selv])
        gy = plsc.load_gather(y_v, [selv])
        gz = plsc.load_gather(z_v, [selv])
        row = jnp.where(
            eq0, plsc.bitcast(mv, jnp.int32),
            jnp.where(
                eq1, selv,
                jnp.where(
                    eq2, plsc.bitcast(gx, jnp.int32),
                    jnp.where(eq3, plsc.bitcast(gy, jnp.int32),
                              plsc.bitcast(gz, jnp.int32)))))
        comm_v[...] = row
        parity = jnp.bitwise_and(k, 1)
        pltpu.sync_copy(comm_v, sh_s.at[pl.ds((parity * 16 + s) * L, L)])
        plsc.subcore_barrier()
        pltpu.sync_copy(sh_s.at[pl.ds((parity * 16 + partner) * L, L)], pr_v)
        pmv = plsc.bitcast(plsc.load_gather(pr_v, [zero_idx]), jnp.float32)
        pselv = plsc.load_gather(pr_v, [one_idx])
        pgx = plsc.bitcast(plsc.load_gather(pr_v, [two_idx]), jnp.float32)
        pgy = plsc.bitcast(plsc.load_gather(pr_v, [three_idx]), jnp.float32)
        pgz = plsc.bitcast(plsc.load_gather(pr_v, [four_idx]), jnp.float32)
        use_mine = jnp.logical_or(
            mv > pmv, jnp.logical_and(mv == pmv, selv < pselv)
        )
        lx = jnp.where(use_mine, gx, pgx)
        ly = jnp.where(use_mine, gy, pgy)
        lz = jnp.where(use_mine, gz, pgz)
        kv = jnp.full((L,), k, jnp.int32)
        plsc.store_scatter(ox_v, [kv], lx, mask=lane0)
        plsc.store_scatter(oy_v, [kv], ly, mask=lane0)
        plsc.store_scatter(oz_v, [kv], lz, mask=lane0)
        return lx, ly, lz

    lax.fori_loop(1, K, outer, (lx, ly, lz), unroll=2)

    @pl.when(half == 0)
    def _():
        pltpu.sync_copy(ox_v, out_hbm.at[pl.ds((b * 3 + 0) * K, K)])
        pltpu.sync_copy(oy_v, out_hbm.at[pl.ds((b * 3 + 1) * K, K)])
        pltpu.sync_copy(oz_v, out_hbm.at[pl.ds((b * 3 + 2) * K, K)])


def kernel(points):
    mesh = plsc.VectorSubcoreMesh(core_axis_name="c", subcore_axis_name="s")
    fn = pl.kernel(
        _fps_body,
        mesh=mesh,
        out_type=jax.ShapeDtypeStruct((B * 3 * K,), jnp.float32),
        compiler_params=pltpu.CompilerParams(needs_layout_passes=False),
        scratch_types=[
            pltpu.VMEM((N,), jnp.float32),
            pltpu.VMEM((N,), jnp.float32),
            pltpu.VMEM((N,), jnp.float32),
            pltpu.VMEM((HALF,), jnp.float32),
            pltpu.VMEM((K,), jnp.float32),
            pltpu.VMEM((K,), jnp.float32),
            pltpu.VMEM((K,), jnp.float32),
            pltpu.VMEM((L,), jnp.int32),
            pltpu.VMEM((L,), jnp.int32),
            pltpu.VMEM_SHARED((2 * 16 * L,), jnp.int32),
        ],
    )
    return fn(points.reshape(B * 3 * N)).reshape(B, 3, K)


# speedup vs baseline: 1.0007x; 1.0007x over previous
import jax
import jax.numpy as jnp
from jax import lax
from jax.experimental import pallas as pl
from jax.experimental.pallas import tpu as pltpu
from jax.experimental.pallas import tpu_sc as plsc

B = 16
N = 16384
K = 2048
L = 16
HALF = N // 2
HCHUNKS = HALF // L


def _fps_body(points_hbm, out_hbm,
              x_v, y_v, z_v, d_v, ox_v, oy_v, oz_v, comm_v, pr_v, sh_s):
    c = lax.axis_index("c")
    s = lax.axis_index("s")
    b = c * 8 + s // 2
    half = s % 2
    partner = s ^ 1
    hbase = half * HALF

    pltpu.sync_copy(points_hbm.at[pl.ds((b * 3 + 0) * N, N)], x_v)
    pltpu.sync_copy(points_hbm.at[pl.ds((b * 3 + 1) * N, N)], y_v)
    pltpu.sync_copy(points_hbm.at[pl.ds((b * 3 + 2) * N, N)], z_v)

    lanes = lax.iota(jnp.int32, L)
    lane0 = lanes == 0
    big = jnp.full((L,), 1e10, jnp.float32)

    @plsc.parallel_loop(0, HCHUNKS, unroll=8)
    def _init(i):
        d_v[pl.ds(i * L, L)] = big

    zero_idx = jnp.full((L,), jnp.min(lanes), jnp.int32)
    lx = plsc.load_gather(x_v, [zero_idx])
    ly = plsc.load_gather(y_v, [zero_idx])
    lz = plsc.load_gather(z_v, [zero_idx])
    plsc.store_scatter(ox_v, [zero_idx], lx, mask=lane0)
    plsc.store_scatter(oy_v, [zero_idx], ly, mask=lane0)
    plsc.store_scatter(oz_v, [zero_idx], lz, mask=lane0)

    eq0 = lanes == 0
    eq1 = lanes == 1
    eq2 = lanes == 2
    eq3 = lanes == 3
    one_idx = jnp.full((L,), 1, jnp.int32)
    two_idx = jnp.full((L,), 2, jnp.int32)
    three_idx = jnp.full((L,), 3, jnp.int32)
    four_idx = jnp.full((L,), 4, jnp.int32)

    def outer(k, carry):
        lx, ly, lz = carry

        def inner(i, mcarry):
            run_max, run_idx = mcarry
            base = i * L
            xv = x_v[pl.ds(hbase + base, L)]
            yv = y_v[pl.ds(hbase + base, L)]
            zv = z_v[pl.ds(hbase + base, L)]
            dv = d_v[pl.ds(base, L)]
            dx = xv - lx
            dy = yv - ly
            dz = zv - lz
            dist = (dx * dx + dz * dz) + dy * dy
            nd = jnp.minimum(dv, dist)
            d_v[pl.ds(base, L)] = nd
            upd = nd > run_max
            run_max = jnp.where(upd, nd, run_max)
            run_idx = jnp.where(upd, hbase + base + lanes, run_idx)
            return run_max, run_idx

        run_max0 = jnp.full((L,), -jnp.inf, jnp.float32)
        run_idx0 = jnp.zeros((L,), jnp.int32)
        run_max, run_idx = plsc.parallel_loop(
            0, HCHUNKS, carry=(run_max0, run_idx0), unroll=8
        )(inner)
        m = jnp.max(run_max)
        sel = jnp.min(jnp.where(run_max == m, run_idx, N))
        mv = jnp.full((L,), m, jnp.float32)
        selv = jnp.full((L,), sel, jnp.int32)
        gx = plsc.load_gather(x_v, [selv])
        gy = plsc.load_gather(y_v, [selv])
        gz = plsc.load_gather(z_v, [selv])
        row = jnp.where(
            eq0, plsc.bitcast(mv, jnp.int32),
            jnp.where(
                eq1, selv,
                jnp.where(
                    eq2, plsc.bitcast(gx, jnp.int32),
                    jnp.where(eq3, plsc.bitcast(gy, jnp.int32),
                              plsc.bitcast(gz, jnp.int32)))))
        comm_v[...] = row
        parity = jnp.bitwise_and(k, 1)
        pltpu.sync_copy(comm_v, sh_s.at[pl.ds((parity * 16 + s) * L, L)])
        plsc.subcore_barrier()
        pltpu.sync_copy(sh_s.at[pl.ds((parity * 16 + partner) * L, L)], pr_v)
        pmv = plsc.bitcast(plsc.load_gather(pr_v, [zero_idx]), jnp.float32)
        pselv = plsc.load_gather(pr_v, [one_idx])
        pgx = plsc.bitcast(plsc.load_gather(pr_v, [two_idx]), jnp.float32)
        pgy = plsc.bitcast(plsc.load_gather(pr_v, [three_idx]), jnp.float32)
        pgz = plsc.bitcast(plsc.load_gather(pr_v, [four_idx]), jnp.float32)
        use_mine = jnp.logical_or(
            mv > pmv, jnp.logical_and(mv == pmv, selv < pselv)
        )
        lx = jnp.where(use_mine, gx, pgx)
        ly = jnp.where(use_mine, gy, pgy)
        lz = jnp.where(use_mine, gz, pgz)
        kv = jnp.full((L,), k, jnp.int32)
        plsc.store_scatter(ox_v, [kv], lx, mask=lane0)
        plsc.store_scatter(oy_v, [kv], ly, mask=lane0)
        plsc.store_scatter(oz_v, [kv], lz, mask=lane0)
        return lx, ly, lz

    lax.fori_loop(1, K, outer, (lx, ly, lz))

    @pl.when(half == 0)
    def _():
        pltpu.sync_copy(ox_v, out_hbm.at[pl.ds((b * 3 + 0) * K, K)])
        pltpu.sync_copy(oy_v, out_hbm.at[pl.ds((b * 3 + 1) * K, K)])
        pltpu.sync_copy(oz_v, out_hbm.at[pl.ds((b * 3 + 2) * K, K)])


def kernel(points):
    mesh = plsc.VectorSubcoreMesh(core_axis_name="c", subcore_axis_name="s")
    fn = pl.kernel(
        _fps_body,
        mesh=mesh,
        out_type=jax.ShapeDtypeStruct((B * 3 * K,), jnp.float32),
        compiler_params=pltpu.CompilerParams(needs_layout_passes=False),
        scratch_types=[
            pltpu.VMEM((N,), jnp.float32),
            pltpu.VMEM((N,), jnp.float32),
            pltpu.VMEM((N,), jnp.float32),
            pltpu.VMEM((HALF,), jnp.float32),
            pltpu.VMEM((K,), jnp.float32),
            pltpu.VMEM((K,), jnp.float32),
            pltpu.VMEM((K,), jnp.float32),
            pltpu.VMEM((L,), jnp.int32),
            pltpu.VMEM((L,), jnp.int32),
            pltpu.VMEM_SHARED((2 * 16 * L,), jnp.int32),
        ],
    )
    return fn(points.reshape(B * 3 * N)).reshape(B, 3, K)
